# Initial kernel scaffold; baseline (speedup 1.0000x reference)
import jax
import jax.numpy as jnp
from jax.experimental import pallas as pl


def kernel(x, edge_index, W0, b0, W1, b1, W2, b2):
    raise NotImplementedError("write your pallas kernel here")


# speedup vs baseline: 4.8788x; 4.8788x over previous
import functools

import jax
import jax.numpy as jnp
from jax import lax
from jax.experimental import pallas as pl
from jax.experimental.pallas import tpu as pltpu
from jax.experimental.pallas import tpu_sc as plsc

N_NODES = 10000
N_EDGES = 320000
N_TILES = 32
EDGES_PER_TILE = N_EDGES // N_TILES
CHUNK = 80
N_CHUNKS = EDGES_PER_TILE // CHUNK
ROWS_PER_TILE = 624
TAIL_BASE = ROWS_PER_TILE * 16
TAIL_ROWS = N_NODES - TAIL_BASE
DEG_W = 16


@functools.lru_cache(maxsize=None)
def _make_agg(D, with_deg):
    mesh = plsc.VectorSubcoreMesh(core_axis_name="c", subcore_axis_name="s")
    outs = jax.ShapeDtypeStruct((2, N_NODES, D), jnp.float32)
    scratch = [
        pltpu.VMEM((CHUNK,), jnp.int32),
        pltpu.VMEM((CHUNK,), jnp.int32),
        pltpu.VMEM((CHUNK, D), jnp.float32),
        pltpu.VMEM_SHARED((N_NODES, D), jnp.float32),
        pltpu.SemaphoreType.DMA,
    ]
    if with_deg:
        outs = [outs, jax.ShapeDtypeStruct((2, N_NODES, DEG_W), jnp.float32)]
        scratch += [
            pltpu.VMEM((CHUNK, DEG_W), jnp.float32),
            pltpu.VMEM_SHARED((N_NODES, DEG_W), jnp.float32),
        ]

    params = pltpu.CompilerParams(use_tc_tiling_on_sc=False)

    @functools.partial(pl.kernel, mesh=mesh, out_type=outs,
                       scratch_types=scratch, compiler_params=params)
    def agg(h_hbm, src_hbm, dst_hbm, z_hbm, *refs):
        if with_deg:
            (zd_hbm, ones_hbm, out_hbm, outd_hbm,
             idx_s, idx_d, rows, acc, sem, ones_v, dacc) = refs
        else:
            (out_hbm, idx_s, idx_d, rows, acc, sem) = refs
        cid = lax.axis_index("c")
        sid = lax.axis_index("s")
        r0 = sid * ROWS_PER_TILE

        def rows_in(src, dst):
            pltpu.sync_copy(src.at[pl.ds(r0, ROWS_PER_TILE)],
                            dst.at[pl.ds(r0, ROWS_PER_TILE)])

            @pl.when(sid == 0)
            def _():
                pltpu.sync_copy(src.at[pl.ds(TAIL_BASE, TAIL_ROWS)],
                                dst.at[pl.ds(TAIL_BASE, TAIL_ROWS)])

        def rows_out(src, dst):
            pltpu.sync_copy(src.at[pl.ds(r0, ROWS_PER_TILE)],
                            dst.at[cid, pl.ds(r0, ROWS_PER_TILE)])

            @pl.when(sid == 0)
            def _():
                pltpu.sync_copy(src.at[pl.ds(TAIL_BASE, TAIL_ROWS)],
                                dst.at[cid, pl.ds(TAIL_BASE, TAIL_ROWS)])

        rows_in(z_hbm, acc)
        if with_deg:
            rows_in(zd_hbm, dacc)
            pltpu.sync_copy(ones_hbm, ones_v)
        plsc.subcore_barrier()

        edge_base = (cid * 16 + sid) * EDGES_PER_TILE

        def body(i, carry):
            base = edge_base + i * CHUNK
            pltpu.sync_copy(src_hbm.at[pl.ds(base, CHUNK)], idx_s)
            pltpu.sync_copy(dst_hbm.at[pl.ds(base, CHUNK)], idx_d)
            pltpu.async_copy(h_hbm.at[idx_s], rows, sem).wait()
            pltpu.sync_copy(rows, acc.at[idx_d], add=True)
            if with_deg:
                pltpu.sync_copy(ones_v, dacc.at[idx_d], add=True)
            return carry

        lax.fori_loop(0, N_CHUNKS, body, 0)
        plsc.subcore_barrier()
        rows_out(acc, out_hbm)
        if with_deg:
            rows_out(dacc, outd_hbm)

    return agg


_BM = 400
_GRID = N_NODES // _BM


def _part_spec(bm, d):
    return [
        pl.BlockSpec((1, bm, d), lambda i: (0, i, 0)),
        pl.BlockSpec((1, bm, d), lambda i: (1, i, 0)),
    ]


def _full_spec(shape):
    nd = len(shape)
    return pl.BlockSpec(shape, lambda i: (0,) * nd)


def _inv_deg(d0_ref, d1_ref):
    deg = d0_ref[0, :, 0:1] + d1_ref[0, :, 0:1]
    return 1.0 / jnp.maximum(deg, 1.0)


def _tc_a_body(a0, a1, d0, d1, w, b, h1a, h1b):
    inv = _inv_deg(d0, d1)
    m = (a0[0] + a1[0]) * inv
    h = jnp.dot(m, w[...], preferred_element_type=jnp.float32) + b[...]
    h = jnp.maximum(h, 0.0)
    h1a[...] = h[:, :128]
    h1b[...] = h[:, 128:]


def _tc_a(aggx, degp, W0, b0):
    return pl.pallas_call(
        _tc_a_body,
        grid=(_GRID,),
        in_specs=_part_spec(_BM, 128) + _part_spec(_BM, DEG_W)
        + [_full_spec((128, 256)), _full_spec((1, 256))],
        out_specs=[pl.BlockSpec((_BM, 128), lambda i: (i, 0))] * 2,
        out_shape=[jax.ShapeDtypeStruct((N_NODES, 128), jnp.float32)] * 2,
    )(aggx, aggx, degp, degp, W0, b0)


def _tc_b_body(l0, l1, r0, r1, d0, d1, w1a, w1b, b, w2, out):
    inv = _inv_deg(d0, d1)
    ml = (l0[0] + l1[0]) * inv
    mr = (r0[0] + r1[0]) * inv
    h = (jnp.dot(ml, w1a[...], preferred_element_type=jnp.float32)
         + jnp.dot(mr, w1b[...], preferred_element_type=jnp.float32)
         + b[...])
    h = jnp.maximum(h, 0.0)
    out[...] = jnp.dot(h, w2[...], preferred_element_type=jnp.float32)


def _tc_b(aggl, aggr, degp, W1a, W1b, b1, W2):
    return pl.pallas_call(
        _tc_b_body,
        grid=(_GRID,),
        in_specs=_part_spec(_BM, 128) * 2 + _part_spec(_BM, DEG_W)
        + [_full_spec((128, 256)), _full_spec((128, 256)),
           _full_spec((1, 256)), _full_spec((256, 64))],
        out_specs=pl.BlockSpec((_BM, 64), lambda i: (i, 0)),
        out_shape=jax.ShapeDtypeStruct((N_NODES, 64), jnp.float32),
    )(aggl, aggl, aggr, aggr, degp, degp, W1a, W1b, b1, W2)


def _tc_c_body(a0, a1, d0, d1, b, out):
    inv = _inv_deg(d0, d1)
    out[...] = (a0[0] + a1[0]) * inv + b[...]


def _tc_c(agg2, degp, b2):
    return pl.pallas_call(
        _tc_c_body,
        grid=(_GRID,),
        in_specs=_part_spec(_BM, 64) + _part_spec(_BM, DEG_W)
        + [_full_spec((1, 64))],
        out_specs=pl.BlockSpec((_BM, 64), lambda i: (i, 0)),
        out_shape=jax.ShapeDtypeStruct((N_NODES, 64), jnp.float32),
    )(agg2, agg2, degp, degp, b2)


def kernel(x, edge_index, W0, b0, W1, b1, W2, b2):
    src = edge_index[0]
    dst = edge_index[1]
    z128 = jnp.zeros((N_NODES, 128), jnp.float32)
    z64 = jnp.zeros((N_NODES, 64), jnp.float32)
    zd = jnp.zeros((N_NODES, DEG_W), jnp.float32)
    ones = jnp.ones((CHUNK, DEG_W), jnp.float32)

    aggx, degp = _make_agg(128, True)(x, src, dst, z128, zd, ones)
    h1a, h1b = _tc_a(aggx, degp, W0, b0.reshape(1, -1))
    aggl = _make_agg(128, False)(h1a, src, dst, z128)
    aggr = _make_agg(128, False)(h1b, src, dst, z128)
    hw2 = _tc_b(aggl, aggr, degp, W1[:128], W1[128:], b1.reshape(1, -1), W2)
    agg2 = _make_agg(64, False)(hw2, src, dst, z64)
    return _tc_c(agg2, degp, b2.reshape(1, -1))


# speedup vs baseline: 8.7263x; 1.7886x over previous
import functools

import jax
import jax.numpy as jnp
from jax import lax
from jax.experimental import pallas as pl
from jax.experimental.pallas import tpu as pltpu
from jax.experimental.pallas import tpu_sc as plsc

N_NODES = 10000
N_EDGES = 320000
N_TILES = 32
CHUNK = 80
N_CHUNKS = N_EDGES // N_TILES // CHUNK
NBUF = 2
ROWS_PER_TILE = 624
TAIL_BASE = ROWS_PER_TILE * 16
TAIL_ROWS = N_NODES - TAIL_BASE
DEG_W = 8


@functools.lru_cache(maxsize=None)
def _make_agg(D, with_deg):
    mesh = plsc.VectorSubcoreMesh(core_axis_name="c", subcore_axis_name="s")
    outs = jax.ShapeDtypeStruct((2, N_NODES, D), jnp.float32)
    scratch = [
        pltpu.VMEM((N_CHUNKS, CHUNK), jnp.int32),
        pltpu.VMEM((N_CHUNKS, CHUNK), jnp.int32),
        pltpu.VMEM_SHARED((N_NODES, D), jnp.float32),
    ]
    scratch += [pltpu.VMEM((CHUNK, D), jnp.float32) for _ in range(NBUF)]
    scratch += [pltpu.SemaphoreType.DMA for _ in range(NBUF)]
    if with_deg:
        outs = [outs, jax.ShapeDtypeStruct((2, N_NODES, DEG_W), jnp.float32)]
        scratch += [
            pltpu.VMEM((CHUNK, DEG_W), jnp.float32),
            pltpu.VMEM_SHARED((N_NODES, DEG_W), jnp.float32),
        ]

    params = pltpu.CompilerParams(use_tc_tiling_on_sc=False)

    @functools.partial(pl.kernel, mesh=mesh, out_type=outs,
                       scratch_types=scratch, compiler_params=params)
    def agg(h_hbm, src_hbm, dst_hbm, z_hbm, *refs):
        if with_deg:
            (zd_hbm, ones_hbm, out_hbm, outd_hbm,
             idx_s, idx_d, acc, *rest) = refs
            ones_v, dacc = rest[-2:]
            rest = rest[:-2]
        else:
            (out_hbm, idx_s, idx_d, acc, *rest) = refs
        rows = rest[:NBUF]
        sems = rest[NBUF:2 * NBUF]
        cid = lax.axis_index("c")
        sid = lax.axis_index("s")
        r0 = sid * ROWS_PER_TILE

        def rows_in(src, dst):
            pltpu.sync_copy(src.at[pl.ds(r0, ROWS_PER_TILE)],
                            dst.at[pl.ds(r0, ROWS_PER_TILE)])

            @pl.when(sid == 0)
            def _():
                pltpu.sync_copy(src.at[pl.ds(TAIL_BASE, TAIL_ROWS)],
                                dst.at[pl.ds(TAIL_BASE, TAIL_ROWS)])

        def rows_out(src, dst):
            pltpu.sync_copy(src.at[pl.ds(r0, ROWS_PER_TILE)],
                            dst.at[cid, pl.ds(r0, ROWS_PER_TILE)])

            @pl.when(sid == 0)
            def _():
                pltpu.sync_copy(src.at[pl.ds(TAIL_BASE, TAIL_ROWS)],
                                dst.at[cid, pl.ds(TAIL_BASE, TAIL_ROWS)])

        crow0 = (cid * 16 + sid) * N_CHUNKS
        pltpu.sync_copy(src_hbm.at[pl.ds(crow0, N_CHUNKS)], idx_s)
        pltpu.sync_copy(dst_hbm.at[pl.ds(crow0, N_CHUNKS)], idx_d)
        rows_in(z_hbm, acc)
        if with_deg:
            rows_in(zd_hbm, dacc)
            pltpu.sync_copy(ones_hbm, ones_v)
        plsc.subcore_barrier()

        def chunk_pair(c0):
            copies = [
                pltpu.async_copy(h_hbm.at[idx_s.at[c0 + b]], rows[b],
                                 sems[b])
                for b in range(NBUF)
            ]
            for b in range(NBUF):
                copies[b].wait()
                pltpu.sync_copy(rows[b], acc.at[idx_d.at[c0 + b]], add=True)
                if with_deg:
                    pltpu.sync_copy(ones_v, dacc.at[idx_d.at[c0 + b]],
                                    add=True)

        def body(g, carry):
            chunk_pair(g * NBUF)
            return carry

        lax.fori_loop(0, N_CHUNKS // NBUF, body, 0)
        for c in range(N_CHUNKS - N_CHUNKS % NBUF, N_CHUNKS):
            pltpu.async_copy(h_hbm.at[idx_s.at[c]], rows[0], sems[0]).wait()
            pltpu.sync_copy(rows[0], acc.at[idx_d.at[c]], add=True)
            if with_deg:
                pltpu.sync_copy(ones_v, dacc.at[idx_d.at[c]], add=True)
        plsc.subcore_barrier()
        rows_out(acc, out_hbm)
        if with_deg:
            rows_out(dacc, outd_hbm)

    return agg


_BM = 400
_GRID = N_NODES // _BM


def _part_spec(bm, d):
    return [
        pl.BlockSpec((1, bm, d), lambda i: (0, i, 0)),
        pl.BlockSpec((1, bm, d), lambda i: (1, i, 0)),
    ]


def _full_spec(shape):
    nd = len(shape)
    return pl.BlockSpec(shape, lambda i: (0,) * nd)


def _inv_deg(d0_ref, d1_ref):
    deg = d0_ref[0, :, 0:1] + d1_ref[0, :, 0:1]
    return 1.0 / jnp.maximum(deg, 1.0)


def _tc_a_body(a0, a1, d0, d1, w, b, h1a, h1b):
    inv = _inv_deg(d0, d1)
    m = (a0[0] + a1[0]) * inv
    h = jnp.dot(m, w[...], preferred_element_type=jnp.float32) + b[...]
    h = jnp.maximum(h, 0.0)
    h1a[...] = h[:, :128]
    h1b[...] = h[:, 128:]


def _tc_a(aggx, degp, W0, b0):
    return pl.pallas_call(
        _tc_a_body,
        grid=(_GRID,),
        in_specs=_part_spec(_BM, 128) + _part_spec(_BM, DEG_W)
        + [_full_spec((128, 256)), _full_spec((1, 256))],
        out_specs=[pl.BlockSpec((_BM, 128), lambda i: (i, 0))] * 2,
        out_shape=[jax.ShapeDtypeStruct((N_NODES, 128), jnp.float32)] * 2,
    )(aggx, aggx, degp, degp, W0, b0)


def _tc_b_body(l0, l1, r0, r1, d0, d1, w1a, w1b, b, w2, out):
    inv = _inv_deg(d0, d1)
    ml = (l0[0] + l1[0]) * inv
    mr = (r0[0] + r1[0]) * inv
    h = (jnp.dot(ml, w1a[...], preferred_element_type=jnp.float32)
         + jnp.dot(mr, w1b[...], preferred_element_type=jnp.float32)
         + b[...])
    h = jnp.maximum(h, 0.0)
    out[...] = jnp.dot(h, w2[...], preferred_element_type=jnp.float32)


def _tc_b(aggl, aggr, degp, W1a, W1b, b1, W2):
    return pl.pallas_call(
        _tc_b_body,
        grid=(_GRID,),
        in_specs=_part_spec(_BM, 128) * 2 + _part_spec(_BM, DEG_W)
        + [_full_spec((128, 256)), _full_spec((128, 256)),
           _full_spec((1, 256)), _full_spec((256, 64))],
        out_specs=pl.BlockSpec((_BM, 64), lambda i: (i, 0)),
        out_shape=jax.ShapeDtypeStruct((N_NODES, 64), jnp.float32),
    )(aggl, aggl, aggr, aggr, degp, degp, W1a, W1b, b1, W2)


def _tc_c_body(a0, a1, d0, d1, b, out):
    inv = _inv_deg(d0, d1)
    out[...] = (a0[0] + a1[0]) * inv + b[...]


def _tc_c(agg2, degp, b2):
    return pl.pallas_call(
        _tc_c_body,
        grid=(_GRID,),
        in_specs=_part_spec(_BM, 64) + _part_spec(_BM, DEG_W)
        + [_full_spec((1, 64))],
        out_specs=pl.BlockSpec((_BM, 64), lambda i: (i, 0)),
        out_shape=jax.ShapeDtypeStruct((N_NODES, 64), jnp.float32),
    )(agg2, agg2, degp, degp, b2)


def kernel(x, edge_index, W0, b0, W1, b1, W2, b2):
    e3 = edge_index.reshape(2, N_TILES * N_CHUNKS, CHUNK)
    src = e3[0]
    dst = e3[1]
    z128 = jnp.zeros((N_NODES, 128), jnp.float32)
    z64 = jnp.zeros((N_NODES, 64), jnp.float32)
    zd = jnp.zeros((N_NODES, DEG_W), jnp.float32)
    ones = jnp.ones((CHUNK, DEG_W), jnp.float32)

    aggx, degp = _make_agg(128, True)(x, src, dst, z128, zd, ones)
    h1a, h1b = _tc_a(aggx, degp, W0, b0.reshape(1, -1))
    aggl = _make_agg(128, False)(h1a, src, dst, z128)
    aggr = _make_agg(128, False)(h1b, src, dst, z128)
    hw2 = _tc_b(aggl, aggr, degp, W1[:128], W1[128:], b1.reshape(1, -1), W2)
    agg2 = _make_agg(64, False)(hw2, src, dst, z64)
    return _tc_c(agg2, degp, b2.reshape(1, -1))


# speedup vs baseline: 8.9962x; 1.0309x over previous
import functools

import jax
import jax.numpy as jnp
from jax import lax
from jax.experimental import pallas as pl
from jax.experimental.pallas import tpu as pltpu
from jax.experimental.pallas import tpu_sc as plsc

N_NODES = 10000
N_EDGES = 320000
N_TILES = 32
CHUNK = 40
N_CHUNKS = N_EDGES // N_TILES // CHUNK
NBUF = 4
ROWS_PER_TILE = 624
TAIL_BASE = ROWS_PER_TILE * 16
TAIL_ROWS = N_NODES - TAIL_BASE
DEG_W = 8


@functools.lru_cache(maxsize=None)
def _make_agg(D, with_deg):
    mesh = plsc.VectorSubcoreMesh(core_axis_name="c", subcore_axis_name="s")
    outs = jax.ShapeDtypeStruct((2, N_NODES, D), jnp.float32)
    scratch = [
        pltpu.VMEM((N_CHUNKS, CHUNK), jnp.int32),
        pltpu.VMEM((N_CHUNKS, CHUNK), jnp.int32),
        pltpu.VMEM_SHARED((N_NODES, D), jnp.float32),
    ]
    scratch += [pltpu.VMEM((CHUNK, D), jnp.float32) for _ in range(NBUF)]
    scratch += [pltpu.SemaphoreType.DMA for _ in range(2 * NBUF)]
    if with_deg:
        outs = [outs, jax.ShapeDtypeStruct((2, N_NODES, DEG_W), jnp.float32)]
        scratch += [
            pltpu.VMEM((CHUNK, DEG_W), jnp.float32),
            pltpu.VMEM_SHARED((N_NODES, DEG_W), jnp.float32),
        ]

    params = pltpu.CompilerParams(use_tc_tiling_on_sc=False)

    @functools.partial(pl.kernel, mesh=mesh, out_type=outs,
                       scratch_types=scratch, compiler_params=params)
    def agg(h_hbm, src_hbm, dst_hbm, z_hbm, *refs):
        if with_deg:
            (zd_hbm, ones_hbm, out_hbm, outd_hbm,
             idx_s, idx_d, acc, *rest) = refs
            ones_v, dacc = rest[-2:]
            rest = rest[:-2]
        else:
            (out_hbm, idx_s, idx_d, acc, *rest) = refs
        rows = rest[:NBUF]
        gsems = rest[NBUF:2 * NBUF]
        ssems = rest[2 * NBUF:3 * NBUF]
        cid = lax.axis_index("c")
        sid = lax.axis_index("s")
        r0 = sid * ROWS_PER_TILE

        def rows_in(src, dst):
            pltpu.sync_copy(src.at[pl.ds(r0, ROWS_PER_TILE)],
                            dst.at[pl.ds(r0, ROWS_PER_TILE)])

            @pl.when(sid == 0)
            def _():
                pltpu.sync_copy(src.at[pl.ds(TAIL_BASE, TAIL_ROWS)],
                                dst.at[pl.ds(TAIL_BASE, TAIL_ROWS)])

        def rows_out(src, dst):
            pltpu.sync_copy(src.at[pl.ds(r0, ROWS_PER_TILE)],
                            dst.at[cid, pl.ds(r0, ROWS_PER_TILE)])

            @pl.when(sid == 0)
            def _():
                pltpu.sync_copy(src.at[pl.ds(TAIL_BASE, TAIL_ROWS)],
                                dst.at[cid, pl.ds(TAIL_BASE, TAIL_ROWS)])

        wid = cid * 16 + sid
        pltpu.sync_copy(src_hbm.at[wid], idx_s)
        pltpu.sync_copy(dst_hbm.at[wid], idx_d)
        rows_in(z_hbm, acc)
        if with_deg:
            rows_in(zd_hbm, dacc)
            pltpu.sync_copy(ones_hbm, ones_v)
        plsc.subcore_barrier()

        def chunk_group(c0, nb):
            gets = [
                pltpu.async_copy(h_hbm.at[idx_s.at[c0 + b]], rows[b],
                                 gsems[b])
                for b in range(nb)
            ]
            puts = []
            for b in range(nb):
                gets[b].wait()
                puts.append(pltpu.async_copy(
                    rows[b], acc.at[idx_d.at[c0 + b]], ssems[b], add=True))
                if with_deg:
                    pltpu.sync_copy(ones_v, dacc.at[idx_d.at[c0 + b]],
                                    add=True)
            for p in puts:
                p.wait()

        def body(g, carry):
            chunk_group(g * NBUF, NBUF)
            return carry

        lax.fori_loop(0, N_CHUNKS // NBUF, body, 0)
        rem = N_CHUNKS % NBUF
        if rem:
            chunk_group(N_CHUNKS - rem, rem)
        plsc.subcore_barrier()
        rows_out(acc, out_hbm)
        if with_deg:
            rows_out(dacc, outd_hbm)

    return agg


_BM = 400
_GRID = N_NODES // _BM


def _part_spec(bm, d):
    return [
        pl.BlockSpec((1, bm, d), lambda i: (0, i, 0)),
        pl.BlockSpec((1, bm, d), lambda i: (1, i, 0)),
    ]


def _full_spec(shape):
    nd = len(shape)
    return pl.BlockSpec(shape, lambda i: (0,) * nd)


def _inv_deg(d0_ref, d1_ref):
    deg = d0_ref[0, :, 0:1] + d1_ref[0, :, 0:1]
    return 1.0 / jnp.maximum(deg, 1.0)


def _tc_a_body(a0, a1, d0, d1, w, b, h1a, h1b):
    inv = _inv_deg(d0, d1)
    m = (a0[0] + a1[0]) * inv
    h = jnp.dot(m, w[...], preferred_element_type=jnp.float32) + b[...]
    h = jnp.maximum(h, 0.0)
    h1a[...] = h[:, :128]
    h1b[...] = h[:, 128:]


def _tc_a(aggx, degp, W0, b0):
    return pl.pallas_call(
        _tc_a_body,
        grid=(_GRID,),
        in_specs=_part_spec(_BM, 128) + _part_spec(_BM, DEG_W)
        + [_full_spec((128, 256)), _full_spec((1, 256))],
        out_specs=[pl.BlockSpec((_BM, 128), lambda i: (i, 0))] * 2,
        out_shape=[jax.ShapeDtypeStruct((N_NODES, 128), jnp.float32)] * 2,
    )(aggx, aggx, degp, degp, W0, b0)


def _tc_b_body(l0, l1, r0, r1, d0, d1, w1a, w1b, b, w2, out):
    inv = _inv_deg(d0, d1)
    ml = (l0[0] + l1[0]) * inv
    mr = (r0[0] + r1[0]) * inv
    h = (jnp.dot(ml, w1a[...], preferred_element_type=jnp.float32)
         + jnp.dot(mr, w1b[...], preferred_element_type=jnp.float32)
         + b[...])
    h = jnp.maximum(h, 0.0)
    out[...] = jnp.dot(h, w2[...], preferred_element_type=jnp.float32)


def _tc_b(aggl, aggr, degp, W1a, W1b, b1, W2):
    return pl.pallas_call(
        _tc_b_body,
        grid=(_GRID,),
        in_specs=_part_spec(_BM, 128) * 2 + _part_spec(_BM, DEG_W)
        + [_full_spec((128, 256)), _full_spec((128, 256)),
           _full_spec((1, 256)), _full_spec((256, 64))],
        out_specs=pl.BlockSpec((_BM, 64), lambda i: (i, 0)),
        out_shape=jax.ShapeDtypeStruct((N_NODES, 64), jnp.float32),
    )(aggl, aggl, aggr, aggr, degp, degp, W1a, W1b, b1, W2)


def _tc_c_body(a0, a1, d0, d1, b, out):
    inv = _inv_deg(d0, d1)
    out[...] = (a0[0] + a1[0]) * inv + b[...]


def _tc_c(agg2, degp, b2):
    return pl.pallas_call(
        _tc_c_body,
        grid=(_GRID,),
        in_specs=_part_spec(_BM, 64) + _part_spec(_BM, DEG_W)
        + [_full_spec((1, 64))],
        out_specs=pl.BlockSpec((_BM, 64), lambda i: (i, 0)),
        out_shape=jax.ShapeDtypeStruct((N_NODES, 64), jnp.float32),
    )(agg2, agg2, degp, degp, b2)


def kernel(x, edge_index, W0, b0, W1, b1, W2, b2):
    e4 = edge_index.reshape(2, N_TILES, N_CHUNKS, CHUNK)
    src = e4[0]
    dst = e4[1]
    z128 = jnp.zeros((N_NODES, 128), jnp.float32)
    z64 = jnp.zeros((N_NODES, 64), jnp.float32)
    zd = jnp.zeros((N_NODES, DEG_W), jnp.float32)
    ones = jnp.ones((CHUNK, DEG_W), jnp.float32)

    aggx, degp = _make_agg(128, True)(x, src, dst, z128, zd, ones)
    h1a, h1b = _tc_a(aggx, degp, W0, b0.reshape(1, -1))
    aggl = _make_agg(128, False)(h1a, src, dst, z128)
    aggr = _make_agg(128, False)(h1b, src, dst, z128)
    hw2 = _tc_b(aggl, aggr, degp, W1[:128], W1[128:], b1.reshape(1, -1), W2)
    agg2 = _make_agg(64, False)(hw2, src, dst, z64)
    return _tc_c(agg2, degp, b2.reshape(1, -1))


# speedup vs baseline: 9.0701x; 1.0082x over previous
import functools

import jax
import jax.numpy as jnp
from jax import lax
from jax.experimental import pallas as pl
from jax.experimental.pallas import tpu as pltpu
from jax.experimental.pallas import tpu_sc as plsc

N_NODES = 10000
N_EDGES = 320000
N_TILES = 32
CHUNK = 40
N_CHUNKS = N_EDGES // N_TILES // CHUNK
NBUF = 4
ROWS_PER_TILE = 624
TAIL_BASE = ROWS_PER_TILE * 16
TAIL_ROWS = N_NODES - TAIL_BASE
DEG_W = 8


@functools.lru_cache(maxsize=None)
def _make_agg(D, with_deg, n_tables=1):
    mesh = plsc.VectorSubcoreMesh(core_axis_name="c", subcore_axis_name="s")
    outs = [jax.ShapeDtypeStruct((2, N_NODES, D), jnp.float32)
            for _ in range(n_tables)]
    scratch = [
        pltpu.VMEM((N_CHUNKS, CHUNK), jnp.int32),
        pltpu.VMEM((N_CHUNKS, CHUNK), jnp.int32),
        pltpu.VMEM_SHARED((N_NODES, D), jnp.float32),
    ]
    scratch += [pltpu.VMEM((CHUNK, D), jnp.float32) for _ in range(NBUF)]
    scratch += [pltpu.SemaphoreType.DMA for _ in range(2 * NBUF)]
    if with_deg:
        outs.append(jax.ShapeDtypeStruct((2, N_NODES, DEG_W), jnp.float32))
        scratch += [
            pltpu.VMEM((CHUNK, DEG_W), jnp.float32),
            pltpu.VMEM_SHARED((N_NODES, DEG_W), jnp.float32),
        ]
    if len(outs) == 1:
        outs = outs[0]

    params = pltpu.CompilerParams(use_tc_tiling_on_sc=False)

    @functools.partial(pl.kernel, mesh=mesh, out_type=outs,
                       scratch_types=scratch, compiler_params=params)
    def agg(*args):
        tables = args[:n_tables]
        src_hbm, dst_hbm, z_hbm = args[n_tables:n_tables + 3]
        refs = args[n_tables + 3:]
        if with_deg:
            (zd_hbm, ones_hbm, *refs) = refs
        out_hbms = refs[:n_tables]
        refs = refs[n_tables:]
        if with_deg:
            (outd_hbm, *refs) = refs
            ones_v, dacc = refs[-2:]
            refs = refs[:-2]
        (idx_s, idx_d, acc, *rest) = refs
        rows = rest[:NBUF]
        gsems = rest[NBUF:2 * NBUF]
        ssems = rest[2 * NBUF:3 * NBUF]
        cid = lax.axis_index("c")
        sid = lax.axis_index("s")
        r0 = sid * ROWS_PER_TILE

        def rows_in(src, dst):
            pltpu.sync_copy(src.at[pl.ds(r0, ROWS_PER_TILE)],
                            dst.at[pl.ds(r0, ROWS_PER_TILE)])

            @pl.when(sid == 0)
            def _():
                pltpu.sync_copy(src.at[pl.ds(TAIL_BASE, TAIL_ROWS)],
                                dst.at[pl.ds(TAIL_BASE, TAIL_ROWS)])

        def rows_out(src, dst):
            pltpu.sync_copy(src.at[pl.ds(r0, ROWS_PER_TILE)],
                            dst.at[cid, pl.ds(r0, ROWS_PER_TILE)])

            @pl.when(sid == 0)
            def _():
                pltpu.sync_copy(src.at[pl.ds(TAIL_BASE, TAIL_ROWS)],
                                dst.at[cid, pl.ds(TAIL_BASE, TAIL_ROWS)])

        wid = cid * 16 + sid
        pltpu.sync_copy(src_hbm.at[wid], idx_s)
        pltpu.sync_copy(dst_hbm.at[wid], idx_d)
        if with_deg:
            rows_in(zd_hbm, dacc)
            pltpu.sync_copy(ones_hbm, ones_v)

        def chunk_group(h_hbm, deg_pass, c0, nb):
            gets = [
                pltpu.async_copy(h_hbm.at[idx_s.at[c0 + b]], rows[b],
                                 gsems[b])
                for b in range(nb)
            ]
            puts = []
            for b in range(nb):
                gets[b].wait()
                puts.append(pltpu.async_copy(
                    rows[b], acc.at[idx_d.at[c0 + b]], ssems[b], add=True))
                if deg_pass:
                    pltpu.sync_copy(ones_v, dacc.at[idx_d.at[c0 + b]],
                                    add=True)
            for p in puts:
                p.wait()

        for t in range(n_tables):
            h_hbm = tables[t]
            deg_pass = with_deg and t == 0
            rows_in(z_hbm, acc)
            plsc.subcore_barrier()

            def body(g, carry):
                chunk_group(h_hbm, deg_pass, g * NBUF, NBUF)
                return carry

            lax.fori_loop(0, N_CHUNKS // NBUF, body, 0)
            rem = N_CHUNKS % NBUF
            if rem:
                chunk_group(h_hbm, deg_pass, N_CHUNKS - rem, rem)
            plsc.subcore_barrier()
            rows_out(acc, out_hbms[t])
            if deg_pass:
                rows_out(dacc, outd_hbm)
            if t + 1 < n_tables:
                plsc.subcore_barrier()

    return agg


_BM = 400
_GRID = N_NODES // _BM


def _part_spec(bm, d):
    return [
        pl.BlockSpec((1, bm, d), lambda i: (0, i, 0)),
        pl.BlockSpec((1, bm, d), lambda i: (1, i, 0)),
    ]


def _full_spec(shape):
    nd = len(shape)
    return pl.BlockSpec(shape, lambda i: (0,) * nd)


def _inv_deg(d0_ref, d1_ref):
    deg = d0_ref[0, :, 0:1] + d1_ref[0, :, 0:1]
    return 1.0 / jnp.maximum(deg, 1.0)


def _tc_a_body(a0, a1, d0, d1, w, b, h1a, h1b):
    inv = _inv_deg(d0, d1)
    m = (a0[0] + a1[0]) * inv
    h = jnp.dot(m, w[...], preferred_element_type=jnp.float32) + b[...]
    h = jnp.maximum(h, 0.0)
    h1a[...] = h[:, :128]
    h1b[...] = h[:, 128:]


def _tc_a(aggx, degp, W0, b0):
    return pl.pallas_call(
        _tc_a_body,
        grid=(_GRID,),
        in_specs=_part_spec(_BM, 128) + _part_spec(_BM, DEG_W)
        + [_full_spec((128, 256)), _full_spec((1, 256))],
        out_specs=[pl.BlockSpec((_BM, 128), lambda i: (i, 0))] * 2,
        out_shape=[jax.ShapeDtypeStruct((N_NODES, 128), jnp.float32)] * 2,
    )(aggx, aggx, degp, degp, W0, b0)


def _tc_b_body(l0, l1, r0, r1, d0, d1, w1a, w1b, b, w2, out):
    inv = _inv_deg(d0, d1)
    ml = (l0[0] + l1[0]) * inv
    mr = (r0[0] + r1[0]) * inv
    h = (jnp.dot(ml, w1a[...], preferred_element_type=jnp.float32)
         + jnp.dot(mr, w1b[...], preferred_element_type=jnp.float32)
         + b[...])
    h = jnp.maximum(h, 0.0)
    out[...] = jnp.dot(h, w2[...], preferred_element_type=jnp.float32)


def _tc_b(aggl, aggr, degp, W1a, W1b, b1, W2):
    return pl.pallas_call(
        _tc_b_body,
        grid=(_GRID,),
        in_specs=_part_spec(_BM, 128) * 2 + _part_spec(_BM, DEG_W)
        + [_full_spec((128, 256)), _full_spec((128, 256)),
           _full_spec((1, 256)), _full_spec((256, 64))],
        out_specs=pl.BlockSpec((_BM, 64), lambda i: (i, 0)),
        out_shape=jax.ShapeDtypeStruct((N_NODES, 64), jnp.float32),
    )(aggl, aggl, aggr, aggr, degp, degp, W1a, W1b, b1, W2)


def _tc_c_body(a0, a1, d0, d1, b, out):
    inv = _inv_deg(d0, d1)
    out[...] = (a0[0] + a1[0]) * inv + b[...]


def _tc_c(agg2, degp, b2):
    return pl.pallas_call(
        _tc_c_body,
        grid=(_GRID,),
        in_specs=_part_spec(_BM, 64) + _part_spec(_BM, DEG_W)
        + [_full_spec((1, 64))],
        out_specs=pl.BlockSpec((_BM, 64), lambda i: (i, 0)),
        out_shape=jax.ShapeDtypeStruct((N_NODES, 64), jnp.float32),
    )(agg2, agg2, degp, degp, b2)


def kernel(x, edge_index, W0, b0, W1, b1, W2, b2):
    e4 = edge_index.reshape(2, N_TILES, N_CHUNKS, CHUNK)
    src = e4[0]
    dst = e4[1]
    z128 = jnp.zeros((N_NODES, 128), jnp.float32)
    z64 = jnp.zeros((N_NODES, 64), jnp.float32)
    zd = jnp.zeros((N_NODES, DEG_W), jnp.float32)
    ones = jnp.ones((CHUNK, DEG_W), jnp.float32)

    aggx, degp = _make_agg(128, True)(x, src, dst, z128, zd, ones)
    h1a, h1b = _tc_a(aggx, degp, W0, b0.reshape(1, -1))
    aggl, aggr = _make_agg(128, False, 2)(h1a, h1b, src, dst, z128)
    hw2 = _tc_b(aggl, aggr, degp, W1[:128], W1[128:], b1.reshape(1, -1), W2)
    agg2 = _make_agg(64, False)(hw2, src, dst, z64)
    return _tc_c(agg2, degp, b2.reshape(1, -1))
